# baseline (speedup 1.0000x reference)
import functools

import jax
import jax.numpy as jnp
import numpy as np
from jax import lax
from jax.experimental import pallas as pl
from jax.experimental.pallas import tpu as pltpu
from jax.experimental.pallas import tpu_sc as plsc

EMB = 64
NC = 2
NS = 16
NW = NC * NS
LANES = 16
NBUF = 4
BLK = 40

_PERM = np.concatenate(
    [np.arange(0, 32, 2), np.arange(1, 32, 2),
     np.arange(32, 64, 2), np.arange(33, 64, 2)]
)


def _sc_pool_sums(x_flat, table_bf, B, S):
    b_per_w = B // NW
    idx_per_w = b_per_w * S

    mesh = plsc.VectorSubcoreMesh(core_axis_name="c", subcore_axis_name="s")

    @functools.partial(
        pl.kernel,
        out_type=jax.ShapeDtypeStruct((B, EMB), jnp.float32),
        mesh=mesh,
        compiler_params=pltpu.CompilerParams(
            use_tc_tiling_on_sc=False, needs_layout_passes=False
        ),
        scratch_types=(
            [pltpu.VMEM((idx_per_w,), jnp.int32)]
            + [pltpu.VMEM((S, EMB), jnp.bfloat16) for _ in range(NBUF)]
            + [pltpu.VMEM((b_per_w, EMB), jnp.float32)]
            + [pltpu.SemaphoreType.DMA for _ in range(NBUF)]
        ),
    )
    def k(x_hbm, table_hbm, out_hbm, idx_v, *rest):
        rows = rest[:NBUF]
        sums_v = rest[NBUF]
        sems = rest[NBUF + 1:]

        wid = lax.axis_index("s") * NC + lax.axis_index("c")
        base = wid * idx_per_w
        pltpu.sync_copy(x_hbm.at[pl.ds(base, idx_per_w)], idx_v)

        def gather(s, bq):
            return pltpu.make_async_copy(
                table_hbm.at[idx_v.at[pl.ds(s * S, S)]], rows[bq], sems[bq]
            )

        for bq in range(NBUF):
            gather(bq, bq).start()

        def accumulate(buf_ref, s):
            zero_bf = jnp.zeros((2 * LANES,), jnp.bfloat16)
            zero_f = jnp.zeros((LANES,), jnp.float32)

            def blk_body(blk, carry):
                def row_body(j, bf_carry):
                    a0, a1 = bf_carry
                    r = blk * BLK + j
                    return (
                        a0 + buf_ref[r, pl.ds(0, 2 * LANES)],
                        a1 + buf_ref[r, pl.ds(2 * LANES, 2 * LANES)],
                    )

                a0, a1 = lax.fori_loop(
                    0, BLK, row_body, (zero_bf, zero_bf), unroll=8
                )
                e0, o0 = plsc.unpack(a0, format=plsc.PackFormat.INTERLEAVED)
                e1, o1 = plsc.unpack(a1, format=plsc.PackFormat.INTERLEAVED)
                f0, f1, f2, f3 = carry
                return (f0 + e0, f1 + o0, f2 + e1, f3 + o1)

            acc = lax.fori_loop(
                0, S // BLK, blk_body, (zero_f, zero_f, zero_f, zero_f)
            )
            for c in range(4):
                sums_v[s, pl.ds(c * LANES, LANES)] = acc[c]

        def outer(g, _):
            for bq in range(NBUF):
                s = g * NBUF + bq
                gather(s, bq).wait()
                accumulate(rows[bq], s)
                nxt = s + NBUF

                @pl.when(nxt < b_per_w)
                def _start_next():
                    gather(nxt, bq).start()

            return _

        lax.fori_loop(0, b_per_w // NBUF, outer, 0)
        pltpu.sync_copy(sums_v, out_hbm.at[pl.ds(wid * b_per_w, b_per_w)])

    return k(x_flat, table_bf)


def _tc_epilogue(sums, lens_col, Wt, bp, B):
    BB = 512
    OUTP = Wt.shape[1]

    def body(s_ref, l_ref, w_ref, b_ref, o_ref):
        means = s_ref[...] / l_ref[...]
        rep = jnp.tanh(means)
        o_ref[...] = (
            jnp.dot(rep, w_ref[...], preferred_element_type=jnp.float32)
            + b_ref[...]
        )

    return pl.pallas_call(
        body,
        grid=(B // BB,),
        in_specs=[
            pl.BlockSpec((BB, EMB), lambda i: (i, 0)),
            pl.BlockSpec((BB, 1), lambda i: (i, 0)),
            pl.BlockSpec((EMB, OUTP), lambda i: (0, 0)),
            pl.BlockSpec((1, OUTP), lambda i: (0, 0)),
        ],
        out_specs=pl.BlockSpec((BB, OUTP), lambda i: (i, 0)),
        out_shape=jax.ShapeDtypeStruct((B, OUTP), jnp.float32),
    )(sums, lens_col, Wt, bp)


def kernel(x, lengths, table, W, b):
    B, S = x.shape
    OUT = W.shape[0]
    OUTP = 8

    x_flat = x.reshape(-1)
    table_bf = table.astype(jnp.bfloat16)
    sums = _sc_pool_sums(x_flat, table_bf, B, S)

    lens_col = lengths[1].reshape(B, 1).astype(jnp.float32)
    Wt = jnp.zeros((EMB, OUTP), W.dtype).at[:, :OUT].set(W.T)
    Wt = Wt[_PERM, :]
    bp = jnp.zeros((1, OUTP), b.dtype).at[0, :OUT].set(b)
    logits = _tc_epilogue(sums, lens_col, Wt, bp, B)
    return logits[:, :OUT]


# speedup vs baseline: 1.0255x; 1.0255x over previous
import functools

import jax
import jax.numpy as jnp
from jax import lax
from jax.experimental import pallas as pl
from jax.experimental.pallas import tpu as pltpu
from jax.experimental.pallas import tpu_sc as plsc

EMB = 64
NC = 2
NS = 16
NW = NC * NS
LANES = 16
NBUF = 6


def _sc_pool_sums(x_flat, table, B, S):
    b_per_w = B // NW
    idx_per_w = b_per_w * S
    n_chunks = EMB // LANES

    mesh = plsc.VectorSubcoreMesh(core_axis_name="c", subcore_axis_name="s")

    @functools.partial(
        pl.kernel,
        out_type=jax.ShapeDtypeStruct((B, EMB), jnp.float32),
        mesh=mesh,
        compiler_params=pltpu.CompilerParams(use_tc_tiling_on_sc=False),
        scratch_types=(
            [pltpu.VMEM((idx_per_w,), jnp.int32)]
            + [pltpu.VMEM((S, EMB), jnp.float32) for _ in range(NBUF)]
            + [pltpu.VMEM((b_per_w, EMB), jnp.float32)]
            + [pltpu.SemaphoreType.DMA for _ in range(NBUF)]
        ),
    )
    def k(x_hbm, table_hbm, out_hbm, idx_v, *rest):
        rows = rest[:NBUF]
        sums_v = rest[NBUF]
        sems = rest[NBUF + 1:]

        wid = lax.axis_index("s") * NC + lax.axis_index("c")
        base = wid * idx_per_w
        pltpu.sync_copy(x_hbm.at[pl.ds(base, idx_per_w)], idx_v)

        def gather(s, bq):
            return pltpu.make_async_copy(
                table_hbm.at[idx_v.at[pl.ds(s * S, S)]], rows[bq], sems[bq]
            )

        for bq in range(NBUF):
            gather(bq, bq).start()

        def accumulate(buf_ref, s):
            def acc_body(j, carry):
                return tuple(
                    carry[c] + buf_ref[j, pl.ds(c * LANES, LANES)]
                    for c in range(n_chunks)
                )

            acc = lax.fori_loop(
                0, S, acc_body,
                tuple(jnp.zeros((LANES,), jnp.float32) for _ in range(n_chunks)),
                unroll=8,
            )
            for c in range(n_chunks):
                sums_v[s, pl.ds(c * LANES, LANES)] = acc[c]

        n_outer = b_per_w // NBUF

        def outer(g, _):
            for bq in range(NBUF):
                s = g * NBUF + bq
                gather(s, bq).wait()
                accumulate(rows[bq], s)
                nxt = s + NBUF

                @pl.when(nxt < b_per_w)
                def _start_next():
                    gather(nxt, bq).start()

            return _

        lax.fori_loop(0, n_outer, outer, 0)
        for bq in range(b_per_w % NBUF):
            s = n_outer * NBUF + bq
            gather(s, bq).wait()
            accumulate(rows[bq], s)
        pltpu.sync_copy(sums_v, out_hbm.at[pl.ds(wid * b_per_w, b_per_w)])

    return k(x_flat, table)


def _tc_epilogue(sums, lens_col, Wt, bp, B):
    OUTP = Wt.shape[1]

    def body(s_ref, l_ref, w_ref, b_ref, o_ref):
        means = s_ref[...] / l_ref[...]
        rep = jnp.tanh(means)
        o_ref[...] = (
            jnp.dot(rep, w_ref[...], preferred_element_type=jnp.float32)
            + b_ref[...]
        )

    return pl.pallas_call(
        body,
        out_shape=jax.ShapeDtypeStruct((B, OUTP), jnp.float32),
    )(sums, lens_col, Wt, bp)


def kernel(x, lengths, table, W, b):
    B, S = x.shape
    OUT = W.shape[0]
    OUTP = 8

    x_flat = x.reshape(-1)
    sums = _sc_pool_sums(x_flat, table, B, S)

    lens_col = lengths[1].reshape(B, 1).astype(jnp.float32)
    Wt = jnp.zeros((EMB, OUTP), W.dtype).at[:, :OUT].set(W.T)
    bp = jnp.zeros((1, OUTP), b.dtype).at[0, :OUT].set(b)
    logits = _tc_epilogue(sums, lens_col, Wt, bp, B)
    return logits[:, :OUT]


# speedup vs baseline: 1.0316x; 1.0059x over previous
import functools

import jax
import jax.numpy as jnp
from jax import lax
from jax.experimental import pallas as pl
from jax.experimental.pallas import tpu as pltpu
from jax.experimental.pallas import tpu_sc as plsc

EMB = 64
NC = 2
NS = 16
NW = NC * NS
LANES = 16
NBUF = 3
SPG = 2


def _sc_pool_sums(x_flat, table, B, S):
    b_per_w = B // NW
    idx_per_w = b_per_w * S
    n_chunks = EMB // LANES

    mesh = plsc.VectorSubcoreMesh(core_axis_name="c", subcore_axis_name="s")

    @functools.partial(
        pl.kernel,
        out_type=jax.ShapeDtypeStruct((B, EMB), jnp.float32),
        mesh=mesh,
        compiler_params=pltpu.CompilerParams(use_tc_tiling_on_sc=False),
        scratch_types=(
            [pltpu.VMEM((idx_per_w,), jnp.int32)]
            + [pltpu.VMEM((SPG * S, EMB), jnp.float32) for _ in range(NBUF)]
            + [pltpu.VMEM((b_per_w, EMB), jnp.float32)]
            + [pltpu.SemaphoreType.DMA for _ in range(NBUF)]
        ),
    )
    def k(x_hbm, table_hbm, out_hbm, idx_v, *rest):
        rows = rest[:NBUF]
        sums_v = rest[NBUF]
        sems = rest[NBUF + 1:]
        n_groups = b_per_w // SPG

        wid = lax.axis_index("s") * NC + lax.axis_index("c")
        base = wid * idx_per_w
        pltpu.sync_copy(x_hbm.at[pl.ds(base, idx_per_w)], idx_v)

        def gather(g, bq):
            return pltpu.make_async_copy(
                table_hbm.at[idx_v.at[pl.ds(g * SPG * S, SPG * S)]],
                rows[bq], sems[bq],
            )

        for bq in range(NBUF):
            gather(bq, bq).start()

        def accumulate(buf_ref, g):
            for u in range(SPG):
                def acc_body(j, carry):
                    return tuple(
                        carry[c] + buf_ref[u * S + j, pl.ds(c * LANES, LANES)]
                        for c in range(n_chunks)
                    )

                acc = lax.fori_loop(
                    0, S, acc_body,
                    tuple(jnp.zeros((LANES,), jnp.float32)
                          for _ in range(n_chunks)),
                    unroll=8,
                )
                for c in range(n_chunks):
                    sums_v[g * SPG + u, pl.ds(c * LANES, LANES)] = acc[c]

        n_outer = n_groups // NBUF

        def outer(gg, _):
            for bq in range(NBUF):
                g = gg * NBUF + bq
                gather(g, bq).wait()
                accumulate(rows[bq], g)
                nxt = g + NBUF

                @pl.when(nxt < n_groups)
                def _start_next():
                    gather(nxt, bq).start()

            return _

        lax.fori_loop(0, n_outer, outer, 0)
        for bq in range(n_groups % NBUF):
            g = n_outer * NBUF + bq
            gather(g, bq).wait()
            accumulate(rows[bq], g)
        pltpu.sync_copy(sums_v, out_hbm.at[pl.ds(wid * b_per_w, b_per_w)])

    return k(x_flat, table)


def _tc_epilogue(sums, lens_col, Wt, bp, B):
    OUTP = Wt.shape[1]

    def body(s_ref, l_ref, w_ref, b_ref, o_ref):
        means = s_ref[...] / l_ref[...]
        rep = jnp.tanh(means)
        o_ref[...] = (
            jnp.dot(rep, w_ref[...], preferred_element_type=jnp.float32)
            + b_ref[...]
        )

    return pl.pallas_call(
        body,
        out_shape=jax.ShapeDtypeStruct((B, OUTP), jnp.float32),
    )(sums, lens_col, Wt, bp)


def kernel(x, lengths, table, W, b):
    B, S = x.shape
    OUT = W.shape[0]
    OUTP = 8

    x_flat = x.reshape(-1)
    sums = _sc_pool_sums(x_flat, table, B, S)

    lens_col = lengths[1].reshape(B, 1).astype(jnp.float32)
    Wt = jnp.zeros((EMB, OUTP), W.dtype).at[:, :OUT].set(W.T)
    bp = jnp.zeros((1, OUTP), b.dtype).at[0, :OUT].set(b)
    logits = _tc_epilogue(sums, lens_col, Wt, bp, B)
    return logits[:, :OUT]
